# baseline (speedup 1.0000x reference)
import functools

import jax
import jax.numpy as jnp
from jax import lax
from jax.experimental import pallas as pl
from jax.experimental.pallas import tpu as pltpu
from jax.experimental.pallas import tpu_sc as plsc

_S = 200
_B = 16384
_D = 64
_N = _B * _S

_BL = 128
_JC = 8

_NC = 2
_NS = 16
_NW = _NC * _NS
_WIN = 128

_NCHUNK = 8
_BC = _B // _NCHUNK
_WPK = _BC // _WIN
_KPW = (_S + _NW - 1) // _NW


def _rank_tc(tsT_chunk, chunk):
    def body(ts_ref, out_ref):
        blk = pl.program_id(0)
        u = lax.bitcast_convert_type(ts_ref[...], jnp.int32)
        u = jnp.where(u == jnp.int32(-2147483648), jnp.int32(0), u)
        key = jnp.where(
            u < 0, jnp.bitwise_xor(jnp.bitwise_not(u), jnp.int32(-2147483648)), u)
        b3 = key[None]
        kio = lax.broadcasted_iota(jnp.int32, (_JC, _S, _BL), 1)
        jio = lax.broadcasted_iota(jnp.int32, (_JC, _S, _BL), 0)
        acc = jnp.zeros((_S, _BL), jnp.int32)
        for jb in range(0, _S, _JC):
            a3 = key[jb:jb + _JC][:, None, :]
            jlt = (jio + jb) < kio
            cmp = (a3 < b3) | ((a3 == b3) & jlt)
            acc = acc + jnp.sum(cmp.astype(jnp.int32), axis=0)
        bvec = (chunk * _BC + blk * _BL
                + lax.broadcasted_iota(jnp.int32, (_S, _BL), 1))
        out_ref[...] = bvec * _S + acc

    return pl.pallas_call(
        body,
        grid=(_BC // _BL,),
        in_specs=[pl.BlockSpec((_S, _BL), lambda i: (0, i))],
        out_specs=pl.BlockSpec((_S, _BL), lambda i: (0, i)),
        out_shape=jax.ShapeDtypeStruct((_S, _BC), jnp.int32),
        compiler_params=pltpu.CompilerParams(
            dimension_semantics=("parallel",)),
    )(tsT_chunk)


def _make_scatter(first):
    mesh = plsc.VectorSubcoreMesh(core_axis_name="c", subcore_axis_name="s")
    out_type = (jax.ShapeDtypeStruct((_N, 128), jnp.float32) if first else ())

    @functools.partial(
        pl.kernel,
        out_type=out_type,
        mesh=mesh,
        scratch_types=[
            pltpu.VMEM((_WIN, 128), jnp.float32),
            pltpu.VMEM((_WPK, _WIN), jnp.int32),
            pltpu.SemaphoreType.DMA,
            pltpu.SemaphoreType.DMA,
        ],
        compiler_params=pltpu.CompilerParams(use_tc_tiling_on_sc=True),
    )
    def k(repl_hbm, idx_hbm, out_hbm, repl_v, ib, sem0, sem1):
        wid = lax.axis_index("s") * _NC + lax.axis_index("c")
        for t in range(_KPW):
            krow = wid + t * _NW

            @pl.when(krow < _S)
            def _():
                base = pl.multiple_of(krow * _WPK, 8)
                cp_i = pltpu.async_copy(idx_hbm.at[pl.ds(base, _WPK)], ib, sem0)
                cp_r = pltpu.async_copy(repl_hbm.at[krow], repl_v, sem1)
                cp_i.wait()
                cp_r.wait()

                @pl.loop(0, _WPK)
                def _(m):
                    pltpu.sync_copy(repl_v, out_hbm.at[ib.at[m]])

    return k


_scatter_first = _make_scatter(True)
_scatter_next = _make_scatter(False)


def kernel(ts, pos_emb_table):
    ts = ts.astype(jnp.float32)
    table = pos_emb_table.astype(jnp.float32)
    nonzero = jnp.any(ts != 0.0)
    eff = jnp.where(nonzero, table[:_S], jnp.broadcast_to(table[_S:_S + 1], (_S, _D)))
    eff = jnp.pad(eff, ((0, 0), (0, 128 - _D)))
    repl = jnp.broadcast_to(eff[:, None, :], (_S, _WIN, 128))

    tsT = ts.T
    dest0 = _rank_tc(tsT[:, 0:_BC], 0)
    out2 = _scatter_first(repl, dest0.reshape(_S * _WPK, _WIN))
    out_ref = jax.new_ref(out2)
    for c in range(1, _NCHUNK):
        dest = _rank_tc(tsT[:, c * _BC:(c + 1) * _BC], c)
        _scatter_next(repl, dest.reshape(_S * _WPK, _WIN), out_ref)
    out2 = out_ref[...]
    return out2[:, :_D].reshape(_B, _S, _D)


# speedup vs baseline: 1.0238x; 1.0238x over previous
import functools

import jax
import jax.numpy as jnp
from jax import lax
from jax.experimental import pallas as pl
from jax.experimental.pallas import tpu as pltpu
from jax.experimental.pallas import tpu_sc as plsc

_S = 200
_B = 16384
_D = 64
_N = _B * _S

_BL = 128
_JC = 8

_NC = 2
_NS = 16
_NW = _NC * _NS
_WIN = 128

_NCHUNK = 4
_BC = _B // _NCHUNK
_WPK = _BC // _WIN
_KPW = (_S + _NW - 1) // _NW


def _rank_tc(tsT_chunk, chunk):
    def body(ts_ref, out_ref):
        blk = pl.program_id(0)
        u = lax.bitcast_convert_type(ts_ref[...], jnp.int32)
        u = jnp.where(u == jnp.int32(-2147483648), jnp.int32(0), u)
        key = jnp.where(
            u < 0, jnp.bitwise_xor(jnp.bitwise_not(u), jnp.int32(-2147483648)), u)
        b3 = key[None]
        kio = lax.broadcasted_iota(jnp.int32, (_JC, _S, _BL), 1)
        jio = lax.broadcasted_iota(jnp.int32, (_JC, _S, _BL), 0)
        acc = jnp.zeros((_S, _BL), jnp.int32)
        for jb in range(0, _S, _JC):
            a3 = key[jb:jb + _JC][:, None, :]
            jlt = (jio + jb) < kio
            cmp = (a3 < b3) | ((a3 == b3) & jlt)
            acc = acc + jnp.sum(cmp.astype(jnp.int32), axis=0)
        bvec = (chunk * _BC + blk * _BL
                + lax.broadcasted_iota(jnp.int32, (_S, _BL), 1))
        out_ref[...] = bvec * _S + acc

    return pl.pallas_call(
        body,
        grid=(_BC // _BL,),
        in_specs=[pl.BlockSpec((_S, _BL), lambda i: (0, i))],
        out_specs=pl.BlockSpec((_S, _BL), lambda i: (0, i)),
        out_shape=jax.ShapeDtypeStruct((_S, _BC), jnp.int32),
        compiler_params=pltpu.CompilerParams(
            dimension_semantics=("parallel",)),
    )(tsT_chunk)


def _make_scatter(first):
    mesh = plsc.VectorSubcoreMesh(core_axis_name="c", subcore_axis_name="s")
    out_type = (jax.ShapeDtypeStruct((_N, 128), jnp.float32) if first else ())

    @functools.partial(
        pl.kernel,
        out_type=out_type,
        mesh=mesh,
        scratch_types=[
            pltpu.VMEM((_WIN, 128), jnp.float32),
            pltpu.VMEM((_WIN, 128), jnp.float32),
            pltpu.VMEM((_WPK, _WIN), jnp.int32),
            pltpu.VMEM((_WPK, _WIN), jnp.int32),
            pltpu.SemaphoreType.DMA,
            pltpu.SemaphoreType.DMA,
            pltpu.SemaphoreType.DMA,
            pltpu.SemaphoreType.DMA,
        ],
        compiler_params=pltpu.CompilerParams(use_tc_tiling_on_sc=True),
    )
    def k(repl_hbm, idx_hbm, out_hbm, rv0, rv1, ib0, ib1, sr0, sr1, si0, si1):
        wid = lax.axis_index("s") * _NC + lax.axis_index("c")
        rvs, ibs = (rv0, rv1), (ib0, ib1)
        srs, sis = (sr0, sr1), (si0, si1)

        def fetch(t):
            krow = wid + t * _NW
            base = pl.multiple_of(krow * _WPK, 8)
            pltpu.async_copy(idx_hbm.at[pl.ds(base, _WPK)], ibs[t % 2], sis[t % 2])
            pltpu.async_copy(repl_hbm.at[krow], rvs[t % 2], srs[t % 2])

        def drain(t):
            krow = wid + t * _NW
            base = pl.multiple_of(krow * _WPK, 8)
            pltpu.make_async_copy(
                idx_hbm.at[pl.ds(base, _WPK)], ibs[t % 2], sis[t % 2]).wait()
            pltpu.make_async_copy(repl_hbm.at[krow], rvs[t % 2], srs[t % 2]).wait()

        fetch(0)
        for t in range(_KPW):
            krow = wid + t * _NW

            @pl.when(krow < _S)
            def _(t=t):
                drain(t)
                if t + 1 < _KPW:
                    krow2 = wid + (t + 1) * _NW

                    @pl.when(krow2 < _S)
                    def _():
                        fetch(t + 1)

                @pl.loop(0, _WPK)
                def _(m):
                    pltpu.sync_copy(rvs[t % 2], out_hbm.at[ibs[t % 2].at[m]])

    return k


_scatter_first = _make_scatter(True)
_scatter_next = _make_scatter(False)


def kernel(ts, pos_emb_table):
    ts = ts.astype(jnp.float32)
    table = pos_emb_table.astype(jnp.float32)
    nonzero = jnp.any(ts != 0.0)
    eff = jnp.where(nonzero, table[:_S], jnp.broadcast_to(table[_S:_S + 1], (_S, _D)))
    eff = jnp.pad(eff, ((0, 0), (0, 128 - _D)))
    repl = jnp.broadcast_to(eff[:, None, :], (_S, _WIN, 128))

    tsT = ts.T
    dest0 = _rank_tc(tsT[:, 0:_BC], 0)
    out2 = _scatter_first(repl, dest0.reshape(_S * _WPK, _WIN))
    out_ref = jax.new_ref(out2)
    for c in range(1, _NCHUNK):
        dest = _rank_tc(tsT[:, c * _BC:(c + 1) * _BC], c)
        _scatter_next(repl, dest.reshape(_S * _WPK, _WIN), out_ref)
    out2 = out_ref[...]
    return out2[:, :_D].reshape(_B, _S, _D)
